# baseline (speedup 1.0000x reference)
import functools

import jax
import jax.numpy as jnp
from jax.experimental import pallas as pl
from jax.experimental.pallas import tpu as pltpu

NA_, NT_ = 4096, 4096
ADIM_ = 8
ET_ = 2
STEPS_ = 2
BLK_ = 256
NBLK_ = NA_ // BLK_


def _mpnn_kernel(e_ref, fa_ref, ft_ref, w_ref, w2_ref,
                 a_out, t_out,
                 a_st, t_st, q_st, m_st):
    s = pl.program_id(0)
    i = pl.program_id(1)

    @pl.when(jnp.logical_and(s == 0, i == 0))
    def _init():
        a_st[...] = fa_ref[...]
        t_st[...] = ft_ref[...]

    @pl.when(i == 0)
    def _zero_q():
        q_st[...] = jnp.zeros_like(q_st)

    t = t_st[...]
    s0 = jnp.sum(t[:, 0:1])

    row0 = i * BLK_
    e_i32 = e_ref[...]

    @pl.when(s == 0)
    def _stash():
        m_st[pl.ds(row0, BLK_), :] = e_i32.astype(jnp.int8)

    m = jnp.where(
        s == 0, e_i32.astype(jnp.float32),
        m_st[pl.ds(row0, BLK_), :].astype(jnp.float32))

    p = jnp.dot(m, t, preferred_element_type=jnp.float32,
                precision=jax.lax.Precision.HIGHEST)
    n0 = s0 - p[:, 0:1]
    n1 = p[:, 1:2]
    w0 = w2_ref[0, :, :]
    w1 = w2_ref[1, :, :]
    a_new = a_st[pl.ds(row0, BLK_), :] + n0 * w0 + n1 * w1
    a_st[pl.ds(row0, BLK_), :] = a_new
    a_out[...] = a_new

    q_st[...] += jax.lax.dot_general(
        m, a_new, (((0,), (0,)), ((), ())),
        preferred_element_type=jnp.float32,
        precision=jax.lax.Precision.HIGHEST)

    @pl.when(i == NBLK_ - 1)
    def _finish_step():
        sigma = jnp.sum(a_st[...], axis=0, keepdims=True)
        bw0 = w_ref[0, :, :]
        bw1 = w_ref[1, :, :]
        t_new = (t_st[...]
                 + jnp.dot(sigma, bw0, preferred_element_type=jnp.float32,
                           precision=jax.lax.Precision.HIGHEST)
                 + jnp.dot(q_st[...], bw1 - bw0,
                           preferred_element_type=jnp.float32,
                           precision=jax.lax.Precision.HIGHEST))
        t_st[...] = t_new
        t_out[...] = t_new


@jax.jit
def kernel(inputs, first_a, first_t, Awij, Awij2):
    na, nt = inputs.shape
    adim = first_a.shape[1]
    et = first_t.shape[1]
    grid = (STEPS_, NBLK_)
    out = pl.pallas_call(
        _mpnn_kernel,
        grid=grid,
        in_specs=[
            pl.BlockSpec((BLK_, nt), lambda s, i: (i * (1 - s), 0)),
            pl.BlockSpec((na, adim), lambda s, i: (0, 0)),
            pl.BlockSpec((nt, et), lambda s, i: (0, 0)),
            pl.BlockSpec((et, adim, et), lambda s, i: (0, 0, 0)),
            pl.BlockSpec((et, 1, adim), lambda s, i: (0, 0, 0)),
        ],
        out_specs=[
            pl.BlockSpec((BLK_, adim), lambda s, i: (i * s, 0)),
            pl.BlockSpec((nt, et), lambda s, i: (0, 0)),
        ],
        out_shape=[
            jax.ShapeDtypeStruct((na, adim), jnp.float32),
            jax.ShapeDtypeStruct((nt, et), jnp.float32),
        ],
        scratch_shapes=[
            pltpu.VMEM((na, adim), jnp.float32),
            pltpu.VMEM((nt, et), jnp.float32),
            pltpu.VMEM((nt, adim), jnp.float32),
            pltpu.VMEM((na, nt), jnp.int8),
        ],
        compiler_params=pltpu.CompilerParams(
            dimension_semantics=("arbitrary", "arbitrary"),
        ),
    )(inputs, first_a, first_t, Awij, Awij2)
    return out[0], out[1]


# speedup vs baseline: 5.5405x; 5.5405x over previous
import jax
import jax.numpy as jnp
from jax.experimental import pallas as pl
from jax.experimental.pallas import tpu as pltpu

NA_, NT_ = 4096, 4096
ADIM_ = 8
ET_ = 2
STEPS_ = 2
BLK_ = 256
NBLK_ = NA_ // BLK_


def _hilo(x):
    hi = x.astype(jnp.bfloat16)
    lo = (x - hi.astype(jnp.float32)).astype(jnp.bfloat16)
    return jnp.concatenate([hi, lo], axis=1)


def _mpnn_kernel(e_ref, fa_ref, ft_ref, w_ref, w2_ref,
                 a_out, t_out,
                 a_st, t_st, tc_st, q_st, m_st):
    s = pl.program_id(0)
    i = pl.program_id(1)

    @pl.when(jnp.logical_and(s == 0, i == 0))
    def _init():
        a_st[...] = fa_ref[...]
        t_st[...] = ft_ref[...]

    @pl.when(i == 0)
    def _start_step():
        q_st[...] = jnp.zeros_like(q_st)
        tc_st[...] = _hilo(t_st[...])

    row0 = i * BLK_

    @pl.when(s == 0)
    def _stash():
        m_st[pl.ds(row0, BLK_), :] = e_ref[...].astype(jnp.int8)

    m = m_st[pl.ds(row0, BLK_), :].astype(jnp.bfloat16)

    s0 = jnp.sum(t_st[:, 0:1])

    pb = jnp.dot(m, tc_st[...], preferred_element_type=jnp.float32)
    p = pb[:, 0:ET_] + pb[:, ET_:2 * ET_]
    n0 = s0 - p[:, 0:1]
    n1 = p[:, 1:2]
    w0 = w2_ref[0, :, :]
    w1 = w2_ref[1, :, :]
    a_new = a_st[pl.ds(row0, BLK_), :] + n0 * w0 + n1 * w1
    a_st[pl.ds(row0, BLK_), :] = a_new
    a_out[...] = a_new

    ac = _hilo(a_new)
    qb = jax.lax.dot_general(
        m, ac, (((0,), (0,)), ((), ())),
        preferred_element_type=jnp.float32)
    q_st[...] += qb[:, 0:ADIM_] + qb[:, ADIM_:2 * ADIM_]

    @pl.when(i == NBLK_ - 1)
    def _finish_step():
        sigma = jnp.sum(a_st[...], axis=0, keepdims=True)
        bw0 = w_ref[0, :, :]
        bw1 = w_ref[1, :, :]
        t_new = (t_st[...]
                 + jnp.dot(sigma, bw0, preferred_element_type=jnp.float32,
                           precision=jax.lax.Precision.HIGHEST)
                 + jnp.dot(q_st[...], bw1 - bw0,
                           preferred_element_type=jnp.float32,
                           precision=jax.lax.Precision.HIGHEST))
        t_st[...] = t_new
        t_out[...] = t_new


@jax.jit
def kernel(inputs, first_a, first_t, Awij, Awij2):
    na, nt = inputs.shape
    adim = first_a.shape[1]
    et = first_t.shape[1]
    grid = (STEPS_, NBLK_)
    out = pl.pallas_call(
        _mpnn_kernel,
        grid=grid,
        in_specs=[
            pl.BlockSpec((BLK_, nt), lambda s, i: (i * (1 - s), 0)),
            pl.BlockSpec((na, adim), lambda s, i: (0, 0)),
            pl.BlockSpec((nt, et), lambda s, i: (0, 0)),
            pl.BlockSpec((et, adim, et), lambda s, i: (0, 0, 0)),
            pl.BlockSpec((et, 1, adim), lambda s, i: (0, 0, 0)),
        ],
        out_specs=[
            pl.BlockSpec((BLK_, adim), lambda s, i: (i * s, 0)),
            pl.BlockSpec((nt, et), lambda s, i: (0, 0)),
        ],
        out_shape=[
            jax.ShapeDtypeStruct((na, adim), jnp.float32),
            jax.ShapeDtypeStruct((nt, et), jnp.float32),
        ],
        scratch_shapes=[
            pltpu.VMEM((na, adim), jnp.float32),
            pltpu.VMEM((nt, et), jnp.float32),
            pltpu.VMEM((nt, 2 * et), jnp.bfloat16),
            pltpu.VMEM((nt, adim), jnp.float32),
            pltpu.VMEM((na, nt), jnp.int8),
        ],
        compiler_params=pltpu.CompilerParams(
            dimension_semantics=("arbitrary", "arbitrary"),
        ),
    )(inputs, first_a, first_t, Awij, Awij2)
    return out[0], out[1]


# speedup vs baseline: 8.6634x; 1.5636x over previous
import jax
import jax.numpy as jnp
from jax.experimental import pallas as pl
from jax.experimental.pallas import tpu as pltpu

NA_, NT_ = 4096, 4096
ADIM_ = 8
ET_ = 2
STEPS_ = 2
BLK_ = 256
NBLK_ = NA_ // BLK_


def _mpnn_kernel(e_ref, faT_ref, ftT_ref, w2T_ref, bw0T_ref, dWT_ref,
                 aT_out, tT_out,
                 aT_st, tT_st, tc_st, qT_st, m_st, s0_st):
    s = pl.program_id(0)
    i = pl.program_id(1)

    @pl.when(jnp.logical_and(s == 0, i == 0))
    def _init():
        aT_st[...] = faT_ref[...]
        tT_st[...] = ftT_ref[...]

    @pl.when(i == 0)
    def _start_step():
        qT_st[...] = jnp.zeros_like(qT_st)
        tT = tT_st[...]
        th = tT.astype(jnp.bfloat16)
        tl = (tT - th.astype(jnp.float32)).astype(jnp.bfloat16)
        tc_st[...] = jnp.concatenate([th, tl], axis=0).T
        s0_st[0, 0] = jnp.sum(tT[0:1, :])

    row0 = i * BLK_

    @pl.when(s == 0)
    def _stash():
        m_st[pl.ds(row0, BLK_), :] = e_ref[...].astype(jnp.int8)

    m = m_st[pl.ds(row0, BLK_), :].astype(jnp.bfloat16)

    pb = jnp.dot(m, tc_st[...], preferred_element_type=jnp.float32)
    pT = pb.T
    p2 = pT[0:ET_, :] + pT[ET_:2 * ET_, :]
    n0 = s0_st[0, 0] - p2[0:1, :]
    n1 = p2[1:2, :]
    a_newT = (aT_st[:, pl.ds(row0, BLK_)]
              + w2T_ref[:, 0:1] * n0
              + w2T_ref[:, 1:2] * n1)
    aT_st[:, pl.ds(row0, BLK_)] = a_newT
    aT_out[...] = a_newT

    ah = a_newT.astype(jnp.bfloat16)
    al = (a_newT - ah.astype(jnp.float32)).astype(jnp.bfloat16)
    acT = jnp.concatenate([ah, al], axis=0)
    qT_st[...] += jax.lax.dot_general(
        acT, m, (((1,), (0,)), ((), ())),
        preferred_element_type=jnp.float32)

    @pl.when(i == NBLK_ - 1)
    def _finish_step():
        qs = qT_st[0:ADIM_, :] + qT_st[ADIM_:2 * ADIM_, :]
        sigmaT = jnp.sum(aT_st[...], axis=1, keepdims=True)
        acc = tT_st[...]
        for k in range(ADIM_):
            acc = (acc
                   + dWT_ref[:, k:k + 1] * qs[k:k + 1, :]
                   + bw0T_ref[:, k:k + 1] * sigmaT[k:k + 1, 0:1])
        tT_st[...] = acc
        tT_out[...] = acc


@jax.jit
def kernel(inputs, first_a, first_t, Awij, Awij2):
    na, nt = inputs.shape
    adim = first_a.shape[1]
    et = first_t.shape[1]
    faT = first_a.T
    ftT = first_t.T
    w2T = Awij2[:, 0, :].T
    bw0T = Awij[0].T
    dWT = (Awij[1] - Awij[0]).T
    grid = (STEPS_, NBLK_)
    aT, tT = pl.pallas_call(
        _mpnn_kernel,
        grid=grid,
        in_specs=[
            pl.BlockSpec((BLK_, nt), lambda s, i: (i * (1 - s), 0)),
            pl.BlockSpec((adim, na), lambda s, i: (0, 0)),
            pl.BlockSpec((et, nt), lambda s, i: (0, 0)),
            pl.BlockSpec((adim, et), lambda s, i: (0, 0)),
            pl.BlockSpec((et, adim), lambda s, i: (0, 0)),
            pl.BlockSpec((et, adim), lambda s, i: (0, 0)),
        ],
        out_specs=[
            pl.BlockSpec((adim, BLK_), lambda s, i: (0, i * s)),
            pl.BlockSpec((et, nt), lambda s, i: (0, 0)),
        ],
        out_shape=[
            jax.ShapeDtypeStruct((adim, na), jnp.float32),
            jax.ShapeDtypeStruct((et, nt), jnp.float32),
        ],
        scratch_shapes=[
            pltpu.VMEM((adim, na), jnp.float32),
            pltpu.VMEM((et, nt), jnp.float32),
            pltpu.VMEM((nt, 2 * et), jnp.bfloat16),
            pltpu.VMEM((2 * adim, nt), jnp.float32),
            pltpu.VMEM((na, nt), jnp.int8),
            pltpu.SMEM((1, 1), jnp.float32),
        ],
        compiler_params=pltpu.CompilerParams(
            dimension_semantics=("arbitrary", "arbitrary"),
        ),
    )(inputs, faT, ftT, w2T, bw0T, dWT)
    return aT.T, tT.T
